# baseline (speedup 1.0000x reference)
import functools

import jax
import jax.numpy as jnp
from jax import lax
from jax.experimental import pallas as pl
from jax.experimental.pallas import tpu as pltpu
from jax.experimental.pallas import tpu_sc as plsc

E = 320000
D = 128
NC = 2
NS = 16
L = 16
NW = NC * NS
EPW = E // NW
C = 80
NCHUNK = EPW // C
G = C // L

_mesh = plsc.VectorSubcoreMesh(core_axis_name="c", subcore_axis_name="s")


@functools.partial(
    pl.kernel,
    out_type=jax.ShapeDtypeStruct((E,), jnp.float32),
    mesh=_mesh,
    compiler_params=pltpu.CompilerParams(needs_layout_passes=False),
    scratch_types=[
        pltpu.VMEM((EPW,), jnp.int32),
        pltpu.VMEM((EPW,), jnp.int32),
        pltpu.VMEM((C, D), jnp.float32),
        pltpu.VMEM((C, D), jnp.float32),
        pltpu.VMEM((C, D), jnp.float32),
        pltpu.VMEM((C, D), jnp.float32),
        pltpu.VMEM((EPW,), jnp.float32),
        pltpu.SemaphoreType.DMA,
        pltpu.SemaphoreType.DMA,
        pltpu.SemaphoreType.DMA,
        pltpu.SemaphoreType.DMA,
    ],
)
def _edge_dot(xu_hbm, xa_hbm, src_hbm, dst_hbm, out_hbm,
              sidx, didx, xu_b0, xu_b1, xa_b0, xa_b1, out_v,
              sem_u0, sem_u1, sem_a0, sem_a1):
    wid = lax.axis_index("s") * NC + lax.axis_index("c")
    base = wid * EPW

    pltpu.sync_copy(src_hbm.at[pl.ds(base, EPW)], sidx)
    pltpu.sync_copy(dst_hbm.at[pl.ds(base, EPW)], didx)

    xu_bufs = (xu_b0, xu_b1)
    xa_bufs = (xa_b0, xa_b1)
    sems_u = (sem_u0, sem_u1)
    sems_a = (sem_a0, sem_a1)

    def start(i, b):
        pltpu.async_copy(xu_hbm.at[sidx.at[pl.ds(i * C, C)]], xu_bufs[b], sems_u[b])
        pltpu.async_copy(xa_hbm.at[didx.at[pl.ds(i * C, C)]], xa_bufs[b], sems_a[b])

    def wait(b):
        pltpu.make_async_copy(xu_hbm.at[pl.ds(0, C)], xu_bufs[b], sems_u[b]).wait()
        pltpu.make_async_copy(xa_hbm.at[pl.ds(0, C)], xa_bufs[b], sems_a[b]).wait()

    def compute(i, b):
        xu_buf = xu_bufs[b]
        xa_buf = xa_bufs[b]

        lane = lax.iota(jnp.int32, L)

        def group_body(g, carry):
            e_idx = g * L + lane
            acc = jnp.zeros((L,), jnp.float32)
            for d in range(D):
                dcol = (lane + d) & (D - 1)
                vu = plsc.load_gather(xu_buf, [e_idx, dcol])
                va = plsc.load_gather(xa_buf, [e_idx, dcol])
                acc = acc + vu * va
            out_v[pl.ds(i * C + g * L, L)] = acc
            return carry

        lax.fori_loop(0, G, group_body, 0)

    start(0, 0)

    def pipe_body(it, carry):
        i = 2 * it
        wait(0)
        start(i + 1, 1)
        compute(i, 0)
        wait(1)
        start(i + 2, 0)
        compute(i + 1, 1)
        return carry

    lax.fori_loop(0, (NCHUNK - 1) // 2, pipe_body, 0)
    wait(0)
    compute(NCHUNK - 1, 0)

    pltpu.sync_copy(out_v, out_hbm.at[pl.ds(base, EPW)])


@jax.jit
def kernel(x_user, x_app, edge_label_index):
    src = edge_label_index[0].astype(jnp.int32)
    dst = edge_label_index[1].astype(jnp.int32)
    return _edge_dot(x_user, x_app, src, dst)


# speedup vs baseline: 1.1322x; 1.1322x over previous
import functools

import jax
import jax.numpy as jnp
from jax import lax
from jax.experimental import pallas as pl
from jax.experimental.pallas import tpu as pltpu
from jax.experimental.pallas import tpu_sc as plsc

E = 320000
N = 10000
D = 128
DP = D // 2
NC = 2
NS = 16
L = 16
NW = NC * NS
EPW = E // NW
C = 80
NCHUNK = EPW // C
G = C // L

_mesh = plsc.VectorSubcoreMesh(core_axis_name="c", subcore_axis_name="s")


@functools.partial(
    pl.kernel,
    out_type=jax.ShapeDtypeStruct((E,), jnp.float32),
    mesh=_mesh,
    compiler_params=pltpu.CompilerParams(needs_layout_passes=False, use_tc_tiling_on_sc=False),
    scratch_types=[
        pltpu.VMEM((EPW,), jnp.int32),
        pltpu.VMEM((EPW,), jnp.int32),
        pltpu.VMEM((C, DP), jnp.int32),
        pltpu.VMEM((C, DP), jnp.int32),
        pltpu.VMEM((C, DP), jnp.int32),
        pltpu.VMEM((C, DP), jnp.int32),
        pltpu.VMEM((EPW,), jnp.float32),
        pltpu.SemaphoreType.DMA,
        pltpu.SemaphoreType.DMA,
        pltpu.SemaphoreType.DMA,
        pltpu.SemaphoreType.DMA,
    ],
)
def _edge_dot(xu_hbm, xa_hbm, src_hbm, dst_hbm, out_hbm,
              sidx, didx, xu_b0, xu_b1, xa_b0, xa_b1, out_v,
              sem_u0, sem_u1, sem_a0, sem_a1):
    wid = lax.axis_index("s") * NC + lax.axis_index("c")
    base = wid * EPW

    pltpu.sync_copy(src_hbm.at[pl.ds(base, EPW)], sidx)
    pltpu.sync_copy(dst_hbm.at[pl.ds(base, EPW)], didx)

    xu_bufs = (xu_b0, xu_b1)
    xa_bufs = (xa_b0, xa_b1)
    sems_u = (sem_u0, sem_u1)
    sems_a = (sem_a0, sem_a1)

    def start(i, b):
        pltpu.async_copy(xu_hbm.at[sidx.at[pl.ds(i * C, C)]], xu_bufs[b], sems_u[b])
        pltpu.async_copy(xa_hbm.at[didx.at[pl.ds(i * C, C)]], xa_bufs[b], sems_a[b])

    def wait(b):
        pltpu.make_async_copy(xu_hbm.at[pl.ds(0, C)], xu_bufs[b], sems_u[b]).wait()
        pltpu.make_async_copy(xa_hbm.at[pl.ds(0, C)], xa_bufs[b], sems_a[b]).wait()

    def compute(i, b):
        xu_buf = xu_bufs[b]
        xa_buf = xa_bufs[b]
        lane = lax.iota(jnp.int32, L)

        def group_body(g, carry):
            e_idx = g * L + lane
            acc = jnp.zeros((L,), jnp.float32)
            for p in range(DP):
                pcol = (lane + p) & (DP - 1)
                pu = plsc.load_gather(xu_buf, [e_idx, pcol])
                pa = plsc.load_gather(xa_buf, [e_idx, pcol])
                bu = plsc.bitcast(pu, jnp.bfloat16)
                ba = plsc.bitcast(pa, jnp.bfloat16)
                lo, hi = plsc.unpack(bu * ba, format=plsc.PackFormat.INTERLEAVED)
                acc = acc + lo + hi
            out_v[pl.ds(i * C + g * L, L)] = acc
            return carry

        lax.fori_loop(0, G, group_body, 0)

    start(0, 0)

    def pipe_body(it, carry):
        i = 2 * it
        wait(0)
        start(i + 1, 1)
        compute(i, 0)
        wait(1)
        start(i + 2, 0)
        compute(i + 1, 1)
        return carry

    lax.fori_loop(0, (NCHUNK - 1) // 2, pipe_body, 0)
    wait(0)
    compute(NCHUNK - 1, 0)

    pltpu.sync_copy(out_v, out_hbm.at[pl.ds(base, EPW)])


@jax.jit
def kernel(x_user, x_app, edge_label_index):
    src = edge_label_index[0].astype(jnp.int32)
    dst = edge_label_index[1].astype(jnp.int32)
    xu_p = lax.bitcast_convert_type(
        x_user.astype(jnp.bfloat16).reshape(N, DP, 2), jnp.int32)
    xa_p = lax.bitcast_convert_type(
        x_app.astype(jnp.bfloat16).reshape(N, DP, 2), jnp.int32)
    return _edge_dot(xu_p, xa_p, src, dst)
